# baseline (device time: 195805 ns/iter reference)
import functools

import jax
import jax.numpy as jnp
from jax import lax
from jax.experimental import pallas as pl
from jax.experimental.pallas import tpu as pltpu

N_DEV = 32
B = 2
SQ = 128
D = 512
HQ = 4
DH = 64
HD = HQ * DH
S_GLOBAL = N_DEV * SQ
CW = N_DEV // 2
CCW = N_DEV // 2 - 1


def kernel(x, Wq, Wk, Wv, Wo):
    def body(x_ref, wq_ref, wk_ref, wv_ref, wo_ref, out_ref, k_ref, v_ref,
             kcw_s, kcw_r, vcw_s, vcw_r, kccw_s, kccw_r, vccw_s, vccw_r):
        my = lax.axis_index("i")
        left = lax.rem(my + N_DEV - 1, N_DEV)
        right = lax.rem(my + 1, N_DEV)

        barrier = pltpu.get_barrier_semaphore()
        for nbr in (left, right):
            pl.semaphore_signal(barrier, inc=1, device_id=(nbr,),
                                device_id_type=pl.DeviceIdType.MESH)
        pl.semaphore_wait(barrier, 2)

        d_idx = lax.broadcasted_iota(jnp.int32, (SQ, HD), 1)
        half = lax.rem(d_idx, DH) // 2
        inv = jnp.exp(half.astype(jnp.float32) * (-2.0 / DH * jnp.log(10000.0)))
        posn = lax.broadcasted_iota(jnp.int32, (SQ, HD), 0) + my * SQ
        ang = posn.astype(jnp.float32) * inv
        cos_t = jnp.cos(ang)
        sin_t = jnp.sin(ang)
        even = lax.rem(d_idx, 2) == 0

        def rope(t):
            t_rot = jnp.where(even, -jnp.roll(t, -1, axis=1),
                              jnp.roll(t, 1, axis=1))
            return t * cos_t + t_rot * sin_t

        wk = wk_ref[...].astype(jnp.bfloat16)
        wv = wv_ref[...].astype(jnp.bfloat16)
        row = pl.ds(my * SQ, SQ)
        for b in range(B):
            xb = x_ref[b].astype(jnp.bfloat16)
            k = rope(jnp.dot(xb, wk, preferred_element_type=jnp.float32))
            v = jnp.dot(xb, wv, preferred_element_type=jnp.float32)
            kb16 = k.astype(jnp.bfloat16)
            vb16 = v.astype(jnp.bfloat16)
            for hh in range(HQ):
                k_ref[b, hh, row, :] = kb16[:, hh * DH:(hh + 1) * DH]
                v_ref[b, hh, row, :] = vb16[:, hh * DH:(hh + 1) * DH]

        sent = []

        def start(ref, send_arr, recv_arr, idx, delta, dev):
            blk = pl.ds(lax.rem(my + 2 * N_DEV + delta, N_DEV) * SQ, SQ)
            r = pltpu.make_async_remote_copy(
                src_ref=ref.at[:, :, blk, :],
                dst_ref=ref.at[:, :, blk, :],
                send_sem=send_arr.at[idx],
                recv_sem=recv_arr.at[idx],
                device_id=(dev,),
                device_id_type=pl.DeviceIdType.MESH,
            )
            r.start()
            sent.append(r)
            return r

        desc = {}
        desc["kcw", 0] = start(k_ref, kcw_s, kcw_r, 0, 0, right)
        desc["vcw", 0] = start(v_ref, vcw_s, vcw_r, 0, 0, right)
        desc["kccw", 0] = start(k_ref, kccw_s, kccw_r, 0, 0, left)
        desc["vccw", 0] = start(v_ref, vccw_s, vccw_r, 0, 0, left)

        wq = wq_ref[...].astype(jnp.bfloat16)
        qs = {}
        for b in range(B):
            xb = x_ref[b].astype(jnp.bfloat16)
            q = rope(jnp.dot(xb, wq, preferred_element_type=jnp.float32))
            q16 = q.astype(jnp.bfloat16)
            for hh in range(HQ):
                qs[b, hh] = q16[:, hh * DH:(hh + 1) * DH]

        state = {}
        for b in range(B):
            for hh in range(HQ):
                state[b, hh] = (
                    jnp.full((SQ, 1), -1e30, jnp.float32),
                    jnp.zeros((SQ, 1), jnp.float32),
                    jnp.zeros((SQ, DH), jnp.float32),
                )

        def flash_update(delta):
            blk = pl.ds(lax.rem(my + 2 * N_DEV + delta, N_DEV) * SQ, SQ)
            for b in range(B):
                for hh in range(HQ):
                    m, l, acc = state[b, hh]
                    kb = k_ref[b, hh, blk, :]
                    vb = v_ref[b, hh, blk, :]
                    s = lax.dot_general(
                        qs[b, hh], kb, (((1,), (1,)), ((), ())),
                        preferred_element_type=jnp.float32) * 0.125
                    m_new = jnp.maximum(m, jnp.max(s, axis=1, keepdims=True))
                    p = jnp.exp(s - m_new)
                    corr = jnp.exp(m - m_new)
                    pv = lax.dot_general(
                        p.astype(jnp.bfloat16), vb, (((1,), (0,)), ((), ())),
                        preferred_element_type=jnp.float32)
                    state[b, hh] = (
                        m_new,
                        l * corr + jnp.sum(p, axis=1, keepdims=True),
                        acc * corr + pv,
                    )

        flash_update(0)

        for h in range(CW):
            desc["kcw", h].wait_recv()
            if h + 1 < CW:
                desc["kcw", h + 1] = start(
                    k_ref, kcw_s, kcw_r, h + 1, -(h + 1), right)
            desc["vcw", h].wait_recv()
            if h + 1 < CW:
                desc["vcw", h + 1] = start(
                    v_ref, vcw_s, vcw_r, h + 1, -(h + 1), right)
            if h < CCW:
                desc["kccw", h].wait_recv()
                if h + 1 < CCW:
                    desc["kccw", h + 1] = start(
                        k_ref, kccw_s, kccw_r, h + 1, h + 1, left)
                desc["vccw", h].wait_recv()
                if h + 1 < CCW:
                    desc["vccw", h + 1] = start(
                        v_ref, vccw_s, vccw_r, h + 1, h + 1, left)
            flash_update(-(h + 1))
            if h < CCW:
                flash_update(h + 1)

        wo = wo_ref[...].astype(jnp.bfloat16)
        for b in range(B):
            ctx_b = jnp.concatenate(
                [state[b, hh][2] / state[b, hh][1] for hh in range(HQ)],
                axis=1).astype(jnp.bfloat16)
            out_ref[b] = jnp.dot(ctx_b, wo, preferred_element_type=jnp.float32)

        for r in sent:
            r.wait_send()

        @functools.partial(pl.run_scoped, sem=pltpu.SemaphoreType.REGULAR)
        def _(sem):
            for nbr in (left, right):
                pl.semaphore_signal(sem, inc=1, device_id=(nbr,),
                                    device_id_type=pl.DeviceIdType.MESH)
            pl.semaphore_wait(sem, 2)

    return pl.pallas_call(
        body,
        out_shape=jax.ShapeDtypeStruct((B, SQ, D), jnp.float32),
        in_specs=[pl.BlockSpec(memory_space=pltpu.VMEM)] * 5,
        out_specs=pl.BlockSpec(memory_space=pltpu.VMEM),
        scratch_shapes=[
            pltpu.VMEM((B, HQ, S_GLOBAL, DH), jnp.bfloat16),
            pltpu.VMEM((B, HQ, S_GLOBAL, DH), jnp.bfloat16),
            pltpu.SemaphoreType.DMA((CW,)),
            pltpu.SemaphoreType.DMA((CW,)),
            pltpu.SemaphoreType.DMA((CW,)),
            pltpu.SemaphoreType.DMA((CW,)),
            pltpu.SemaphoreType.DMA((CCW,)),
            pltpu.SemaphoreType.DMA((CCW,)),
            pltpu.SemaphoreType.DMA((CCW,)),
            pltpu.SemaphoreType.DMA((CCW,)),
        ],
        compiler_params=pltpu.CompilerParams(collective_id=0),
    )(x, Wq, Wk, Wv, Wo)


# device time: 112600 ns/iter; 1.7389x vs baseline; 1.7389x over previous
import functools

import jax
import jax.numpy as jnp
from jax import lax
from jax.experimental import pallas as pl
from jax.experimental.pallas import tpu as pltpu

N_DEV = 32
B = 2
SQ = 128
D = 512
HQ = 4
DH = 64
HD = HQ * DH
S_GLOBAL = N_DEV * SQ
CW = N_DEV // 2
CCW = N_DEV // 2 - 1


def kernel(x, Wq, Wk, Wv, Wo):
    def body(x_ref, wq_ref, wk_ref, wv_ref, wo_ref, out_ref, kv_ref, s_ref,
             kcw_s, kcw_r, vcw_s, vcw_r, kccw_s, kccw_r, vccw_s, vccw_r):
        my = lax.axis_index("i")
        left = lax.rem(my + N_DEV - 1, N_DEV)
        right = lax.rem(my + 1, N_DEV)

        barrier = pltpu.get_barrier_semaphore()
        for nbr in (left, right):
            pl.semaphore_signal(barrier, inc=1, device_id=(nbr,),
                                device_id_type=pl.DeviceIdType.MESH)
        pl.semaphore_wait(barrier, 2)

        d_idx = lax.broadcasted_iota(jnp.int32, (SQ, HD), 1)
        half = lax.rem(d_idx, DH) // 2
        inv = jnp.exp(half.astype(jnp.float32) * (-2.0 / DH * jnp.log(10000.0)))
        posn = lax.broadcasted_iota(jnp.int32, (SQ, HD), 0) + my * SQ
        ang = posn.astype(jnp.float32) * inv
        cos_t = jnp.cos(ang)
        sin_t = jnp.sin(ang)
        even = lax.rem(d_idx, 2) == 0

        def rope(t):
            t_rot = jnp.where(even, -jnp.roll(t, -1, axis=1),
                              jnp.roll(t, 1, axis=1))
            return t * cos_t + t_rot * sin_t

        wk = wk_ref[...].astype(jnp.bfloat16)
        wv = wv_ref[...].astype(jnp.bfloat16)
        for b in range(B):
            xb = x_ref[b].astype(jnp.bfloat16)
            k = rope(jnp.dot(xb, wk, preferred_element_type=jnp.float32))
            v = jnp.dot(xb, wv, preferred_element_type=jnp.float32)
            row = pl.ds(my * SQ, SQ)
            kv_ref[b, row, :HD] = k.astype(jnp.bfloat16)
            kv_ref[b, row, HD:] = v.astype(jnp.bfloat16)

        sent = []

        def start(send_arr, recv_arr, idx, delta, col_off, dev):
            blk = pl.ds(lax.rem(my + 2 * N_DEV + delta, N_DEV) * SQ, SQ)
            r = pltpu.make_async_remote_copy(
                src_ref=kv_ref.at[:, blk, pl.ds(col_off, HD)],
                dst_ref=kv_ref.at[:, blk, pl.ds(col_off, HD)],
                send_sem=send_arr.at[idx],
                recv_sem=recv_arr.at[idx],
                device_id=(dev,),
                device_id_type=pl.DeviceIdType.MESH,
            )
            r.start()
            sent.append(r)
            return r

        desc = {}
        desc["kcw", 0] = start(kcw_s, kcw_r, 0, 0, 0, right)
        desc["vcw", 0] = start(vcw_s, vcw_r, 0, 0, HD, right)
        desc["kccw", 0] = start(kccw_s, kccw_r, 0, 0, 0, left)
        desc["vccw", 0] = start(vccw_s, vccw_r, 0, 0, HD, left)

        wq = wq_ref[...].astype(jnp.bfloat16)
        qs = []
        for b in range(B):
            xb = x_ref[b].astype(jnp.bfloat16)
            q = rope(jnp.dot(xb, wq, preferred_element_type=jnp.float32))
            qs.append(q.astype(jnp.bfloat16))

        def score_update(delta):
            blk = pl.ds(lax.rem(my + 2 * N_DEV + delta, N_DEV) * SQ, SQ)
            for b in range(B):
                for hh in range(HQ):
                    kb = kv_ref[b, blk, hh * DH:(hh + 1) * DH]
                    qbh = qs[b][:, hh * DH:(hh + 1) * DH]
                    s = lax.dot_general(
                        qbh, kb, (((1,), (1,)), ((), ())),
                        preferred_element_type=jnp.float32)
                    s_ref[b, hh, :, blk] = s

        score_update(0)

        for h in range(CW):
            desc["kcw", h].wait_recv()
            if h + 1 < CW:
                desc["kcw", h + 1] = start(
                    kcw_s, kcw_r, h + 1, -(h + 1), 0, right)
            desc["vcw", h].wait_recv()
            if h + 1 < CW:
                desc["vcw", h + 1] = start(
                    vcw_s, vcw_r, h + 1, -(h + 1), HD, right)
            if h < CCW:
                desc["kccw", h].wait_recv()
                if h + 1 < CCW:
                    desc["kccw", h + 1] = start(
                        kccw_s, kccw_r, h + 1, h + 1, 0, left)
                desc["vccw", h].wait_recv()
                if h + 1 < CCW:
                    desc["vccw", h + 1] = start(
                        vccw_s, vccw_r, h + 1, h + 1, HD, left)
            score_update(-(h + 1))
            if h < CCW:
                score_update(h + 1)

        wo = wo_ref[...].astype(jnp.bfloat16)
        for b in range(B):
            ctx_heads = []
            for hh in range(HQ):
                s = s_ref[b, hh] * 0.125
                m = jnp.max(s, axis=1, keepdims=True)
                p = jnp.exp(s - m)
                l = jnp.sum(p, axis=1, keepdims=True)
                vf = kv_ref[b, :, HD + hh * DH:HD + (hh + 1) * DH]
                pv = lax.dot_general(
                    p.astype(jnp.bfloat16), vf, (((1,), (0,)), ((), ())),
                    preferred_element_type=jnp.float32)
                ctx_heads.append(pv / l)
            ctx_b = jnp.concatenate(ctx_heads, axis=1).astype(jnp.bfloat16)
            out_ref[b] = jnp.dot(ctx_b, wo, preferred_element_type=jnp.float32)

        for r in sent:
            r.wait_send()

        @functools.partial(pl.run_scoped, sem=pltpu.SemaphoreType.REGULAR)
        def _(sem):
            for nbr in (left, right):
                pl.semaphore_signal(sem, inc=1, device_id=(nbr,),
                                    device_id_type=pl.DeviceIdType.MESH)
            pl.semaphore_wait(sem, 2)

    return pl.pallas_call(
        body,
        out_shape=jax.ShapeDtypeStruct((B, SQ, D), jnp.float32),
        in_specs=[pl.BlockSpec(memory_space=pltpu.VMEM)] * 5,
        out_specs=pl.BlockSpec(memory_space=pltpu.VMEM),
        scratch_shapes=[
            pltpu.VMEM((B, S_GLOBAL, 2 * HD), jnp.bfloat16),
            pltpu.VMEM((B, HQ, SQ, S_GLOBAL), jnp.float32),
            pltpu.SemaphoreType.DMA((CW,)),
            pltpu.SemaphoreType.DMA((CW,)),
            pltpu.SemaphoreType.DMA((CW,)),
            pltpu.SemaphoreType.DMA((CW,)),
            pltpu.SemaphoreType.DMA((CCW,)),
            pltpu.SemaphoreType.DMA((CCW,)),
            pltpu.SemaphoreType.DMA((CCW,)),
            pltpu.SemaphoreType.DMA((CCW,)),
        ],
        compiler_params=pltpu.CompilerParams(collective_id=0),
    )(x, Wq, Wk, Wv, Wo)


# device time: 74423 ns/iter; 2.6310x vs baseline; 1.5130x over previous
import functools

import jax
import jax.numpy as jnp
import numpy as np
from jax import lax
from jax.experimental import pallas as pl
from jax.experimental.pallas import tpu as pltpu

N_DEV = 32
B = 2
SQ = 128
D = 512
HQ = 4
DH = 64
HD = HQ * DH
S_GLOBAL = N_DEV * SQ
CW = N_DEV // 2
CCW = N_DEV // 2 - 1


@functools.lru_cache(maxsize=1)
def _ring_tables():
    import distributed_mesh_v7x as dm

    mesh = dm.get_mesh("i", N_DEV)
    devs = list(mesh.devices.flat)
    coord_to_logical = {tuple(d.coords): i for i, d in enumerate(devs)}
    xs = sorted({c[0] for c in coord_to_logical})
    ys = sorted({c[1] for c in coord_to_logical})
    zs = sorted({c[2] for c in coord_to_logical})
    cycle = None
    if len(xs) == 2 and len(ys) == 4 and len(zs) == 4:
        path_yz = []
        for zi, z in enumerate(zs):
            for y in (ys if zi % 2 == 0 else ys[::-1]):
                path_yz.append((y, z))
        coords = [(xs[0], y, z) for (y, z) in path_yz]
        coords += [(xs[1], y, z) for (y, z) in reversed(path_yz)]
        if all(c in coord_to_logical for c in coords):
            cycle = [coord_to_logical[c] for c in coords]
    if cycle is None:
        cycle = list(range(N_DEV))
    pos = [0] * N_DEV
    for p, l in enumerate(cycle):
        pos[l] = p
    return np.array(cycle, np.int32), np.array(pos, np.int32)


def kernel(x, Wq, Wk, Wv, Wo):
    cycle_np, pos_np = _ring_tables()
    cycle = jnp.asarray(cycle_np)
    pos_of = jnp.asarray(pos_np)
    my = lax.axis_index("i")
    pos = jnp.take(pos_of, my)
    steps = jnp.arange(N_DEV, dtype=jnp.int32)
    ocw = jnp.take(cycle, jnp.remainder(pos - steps, N_DEV)).astype(jnp.int32)
    occw = jnp.take(cycle, jnp.remainder(pos + steps, N_DEV)).astype(jnp.int32)
    nbrs = jnp.stack([jnp.take(cycle, jnp.remainder(pos - 1, N_DEV)),
                      jnp.take(cycle, jnp.remainder(pos + 1, N_DEV))]
                     ).astype(jnp.int32)

    def body(x_ref, wq_ref, wk_ref, wv_ref, wo_ref, nbr_ref, ocw_ref,
             occw_ref, out_ref, kv_ref, s_ref,
             kcw_s, kcw_r, vcw_s, vcw_r, kccw_s, kccw_r, vccw_s, vccw_r):
        my = lax.axis_index("i")
        left = nbr_ref[0]
        right = nbr_ref[1]

        barrier = pltpu.get_barrier_semaphore()
        for nbr in (left, right):
            pl.semaphore_signal(barrier, inc=1, device_id=(nbr,),
                                device_id_type=pl.DeviceIdType.MESH)
        pl.semaphore_wait(barrier, 2)

        d_idx = lax.broadcasted_iota(jnp.int32, (SQ, HD), 1)
        half = lax.rem(d_idx, DH) // 2
        inv = jnp.exp(half.astype(jnp.float32) * (-2.0 / DH * jnp.log(10000.0)))
        posn = lax.broadcasted_iota(jnp.int32, (SQ, HD), 0) + my * SQ
        ang = posn.astype(jnp.float32) * inv
        cos_t = jnp.cos(ang)
        sin_t = jnp.sin(ang)
        even = lax.rem(d_idx, 2) == 0

        def rope(t):
            t_rot = jnp.where(even, -jnp.roll(t, -1, axis=1),
                              jnp.roll(t, 1, axis=1))
            return t * cos_t + t_rot * sin_t

        wk = wk_ref[...].astype(jnp.bfloat16)
        wv = wv_ref[...].astype(jnp.bfloat16)
        for b in range(B):
            xb = x_ref[b].astype(jnp.bfloat16)
            k = rope(jnp.dot(xb, wk, preferred_element_type=jnp.float32))
            v = jnp.dot(xb, wv, preferred_element_type=jnp.float32)
            row = pl.ds(my * SQ, SQ)
            kv_ref[b, row, :HD] = k.astype(jnp.bfloat16)
            kv_ref[b, row, HD:] = v.astype(jnp.bfloat16)

        sent = []

        def start(send_arr, recv_arr, idx, origin, col_off, dev):
            blk = pl.ds(origin * SQ, SQ)
            r = pltpu.make_async_remote_copy(
                src_ref=kv_ref.at[:, blk, pl.ds(col_off, HD)],
                dst_ref=kv_ref.at[:, blk, pl.ds(col_off, HD)],
                send_sem=send_arr.at[idx],
                recv_sem=recv_arr.at[idx],
                device_id=(dev,),
                device_id_type=pl.DeviceIdType.MESH,
            )
            r.start()
            sent.append(r)
            return r

        desc = {}
        desc["kcw", 0] = start(kcw_s, kcw_r, 0, ocw_ref[0], 0, right)
        desc["vcw", 0] = start(vcw_s, vcw_r, 0, ocw_ref[0], HD, right)
        desc["kccw", 0] = start(kccw_s, kccw_r, 0, occw_ref[0], 0, left)
        desc["vccw", 0] = start(vccw_s, vccw_r, 0, occw_ref[0], HD, left)

        wq = wq_ref[...].astype(jnp.bfloat16)
        qs = []
        for b in range(B):
            xb = x_ref[b].astype(jnp.bfloat16)
            q = rope(jnp.dot(xb, wq, preferred_element_type=jnp.float32))
            qs.append(q.astype(jnp.bfloat16))

        def score_update(origin):
            blk = pl.ds(origin * SQ, SQ)
            for b in range(B):
                for hh in range(HQ):
                    kb = kv_ref[b, blk, hh * DH:(hh + 1) * DH]
                    qbh = qs[b][:, hh * DH:(hh + 1) * DH]
                    s = lax.dot_general(
                        qbh, kb, (((1,), (1,)), ((), ())),
                        preferred_element_type=jnp.float32)
                    s_ref[b, hh, :, blk] = s

        score_update(ocw_ref[0])

        for h in range(CW):
            desc["kcw", h].wait_recv()
            if h + 1 < CW:
                desc["kcw", h + 1] = start(
                    kcw_s, kcw_r, h + 1, ocw_ref[h + 1], 0, right)
            desc["vcw", h].wait_recv()
            if h + 1 < CW:
                desc["vcw", h + 1] = start(
                    vcw_s, vcw_r, h + 1, ocw_ref[h + 1], HD, right)
            if h < CCW:
                desc["kccw", h].wait_recv()
                if h + 1 < CCW:
                    desc["kccw", h + 1] = start(
                        kccw_s, kccw_r, h + 1, occw_ref[h + 1], 0, left)
                desc["vccw", h].wait_recv()
                if h + 1 < CCW:
                    desc["vccw", h + 1] = start(
                        vccw_s, vccw_r, h + 1, occw_ref[h + 1], HD, left)
            score_update(ocw_ref[h + 1])
            if h < CCW:
                score_update(occw_ref[h + 1])

        wo = wo_ref[...].astype(jnp.bfloat16)
        for b in range(B):
            ctx_heads = []
            for hh in range(HQ):
                s = s_ref[b, hh] * 0.125
                m = jnp.max(s, axis=1, keepdims=True)
                p = jnp.exp(s - m)
                l = jnp.sum(p, axis=1, keepdims=True)
                vf = kv_ref[b, :, HD + hh * DH:HD + (hh + 1) * DH]
                pv = lax.dot_general(
                    p.astype(jnp.bfloat16), vf, (((1,), (0,)), ((), ())),
                    preferred_element_type=jnp.float32)
                ctx_heads.append(pv / l)
            ctx_b = jnp.concatenate(ctx_heads, axis=1).astype(jnp.bfloat16)
            out_ref[b] = jnp.dot(ctx_b, wo, preferred_element_type=jnp.float32)

        for r in sent:
            r.wait_send()

        @functools.partial(pl.run_scoped, sem=pltpu.SemaphoreType.REGULAR)
        def _(sem):
            for nbr in (left, right):
                pl.semaphore_signal(sem, inc=1, device_id=(nbr,),
                                    device_id_type=pl.DeviceIdType.MESH)
            pl.semaphore_wait(sem, 2)

    return pl.pallas_call(
        body,
        out_shape=jax.ShapeDtypeStruct((B, SQ, D), jnp.float32),
        in_specs=(
            [pl.BlockSpec(memory_space=pltpu.VMEM)] * 5
            + [pl.BlockSpec(memory_space=pltpu.SMEM)] * 3
        ),
        out_specs=pl.BlockSpec(memory_space=pltpu.VMEM),
        scratch_shapes=[
            pltpu.VMEM((B, S_GLOBAL, 2 * HD), jnp.bfloat16),
            pltpu.VMEM((B, HQ, SQ, S_GLOBAL), jnp.float32),
            pltpu.SemaphoreType.DMA((CW,)),
            pltpu.SemaphoreType.DMA((CW,)),
            pltpu.SemaphoreType.DMA((CW,)),
            pltpu.SemaphoreType.DMA((CW,)),
            pltpu.SemaphoreType.DMA((CCW,)),
            pltpu.SemaphoreType.DMA((CCW,)),
            pltpu.SemaphoreType.DMA((CCW,)),
            pltpu.SemaphoreType.DMA((CCW,)),
        ],
        compiler_params=pltpu.CompilerParams(collective_id=0),
    )(x, Wq, Wk, Wv, Wo, nbrs, ocw, occw)


# device time: 74232 ns/iter; 2.6377x vs baseline; 1.0026x over previous
import functools

import jax
import jax.numpy as jnp
import numpy as np
from jax import lax
from jax.experimental import pallas as pl
from jax.experimental.pallas import tpu as pltpu

N_DEV = 32
B = 2
SQ = 128
D = 512
HQ = 4
DH = 64
HD = HQ * DH
S_GLOBAL = N_DEV * SQ
CW = N_DEV // 2
CCW = N_DEV // 2 - 1


@functools.lru_cache(maxsize=1)
def _ring_tables():
    import distributed_mesh_v7x as dm

    mesh = dm.get_mesh("i", N_DEV)
    devs = list(mesh.devices.flat)
    coord_to_logical = {tuple(d.coords): i for i, d in enumerate(devs)}
    xs = sorted({c[0] for c in coord_to_logical})
    ys = sorted({c[1] for c in coord_to_logical})
    zs = sorted({c[2] for c in coord_to_logical})
    cycle = None
    if len(xs) == 2 and len(ys) == 4 and len(zs) == 4:
        path_yz = []
        for zi, z in enumerate(zs):
            for y in (ys if zi % 2 == 0 else ys[::-1]):
                path_yz.append((y, z))
        coords = [(xs[0], y, z) for (y, z) in path_yz]
        coords += [(xs[1], y, z) for (y, z) in reversed(path_yz)]
        if all(c in coord_to_logical for c in coords):
            cycle = [coord_to_logical[c] for c in coords]
    if cycle is None:
        cycle = list(range(N_DEV))
    pos = [0] * N_DEV
    for p, l in enumerate(cycle):
        pos[l] = p
    return np.array(cycle, np.int32), np.array(pos, np.int32)


def kernel(x, Wq, Wk, Wv, Wo):
    cycle_np, pos_np = _ring_tables()
    cycle = jnp.asarray(cycle_np)
    pos_of = jnp.asarray(pos_np)
    my = lax.axis_index("i")
    pos = jnp.take(pos_of, my)
    steps = jnp.arange(N_DEV, dtype=jnp.int32)
    ocw = jnp.take(cycle, jnp.remainder(pos - steps, N_DEV)).astype(jnp.int32)
    occw = jnp.take(cycle, jnp.remainder(pos + steps, N_DEV)).astype(jnp.int32)
    nbrs = jnp.stack([jnp.take(cycle, jnp.remainder(pos - 1, N_DEV)),
                      jnp.take(cycle, jnp.remainder(pos + 1, N_DEV))]
                     ).astype(jnp.int32)

    def body(x_ref, wq_ref, wk_ref, wv_ref, wo_ref, nbr_ref, ocw_ref,
             occw_ref, out_ref, kv_ref, s_ref,
             kcw_s, kcw_r, vcw_s, vcw_r, kccw_s, kccw_r, vccw_s, vccw_r):
        my = lax.axis_index("i")
        left = nbr_ref[0]
        right = nbr_ref[1]

        barrier = pltpu.get_barrier_semaphore()
        for nbr in (left, right):
            pl.semaphore_signal(barrier, inc=1, device_id=(nbr,),
                                device_id_type=pl.DeviceIdType.MESH)
        pl.semaphore_wait(barrier, 2)

        d_idx = lax.broadcasted_iota(jnp.int32, (SQ, HD), 1)
        half = lax.rem(d_idx, DH) // 2
        inv = jnp.exp(half.astype(jnp.float32) * (-2.0 / DH * jnp.log(10000.0)))
        posn = lax.broadcasted_iota(jnp.int32, (SQ, HD), 0) + my * SQ
        ang = posn.astype(jnp.float32) * inv
        cos_t = jnp.cos(ang)
        sin_t = jnp.sin(ang)
        even = lax.rem(d_idx, 2) == 0

        def rope(t):
            t_rot = jnp.where(even, -jnp.roll(t, -1, axis=1),
                              jnp.roll(t, 1, axis=1))
            return t * cos_t + t_rot * sin_t

        wk = wk_ref[...].astype(jnp.bfloat16)
        wv = wv_ref[...].astype(jnp.bfloat16)
        for b in range(B):
            xb = x_ref[b].astype(jnp.bfloat16)
            k = rope(jnp.dot(xb, wk, preferred_element_type=jnp.float32))
            v = jnp.dot(xb, wv, preferred_element_type=jnp.float32)
            row = pl.ds(my * SQ, SQ)
            kv_ref[b, row, :HD] = k.astype(jnp.bfloat16)
            kv_ref[b, row, HD:] = v.astype(jnp.bfloat16)

        sent = []

        def start(send_arr, recv_arr, b, idx, origin, col_off, dev):
            blk = pl.ds(origin * SQ, SQ)
            r = pltpu.make_async_remote_copy(
                src_ref=kv_ref.at[b, blk, pl.ds(col_off, HD)],
                dst_ref=kv_ref.at[b, blk, pl.ds(col_off, HD)],
                send_sem=send_arr.at[b, idx],
                recv_sem=recv_arr.at[b, idx],
                device_id=(dev,),
                device_id_type=pl.DeviceIdType.MESH,
            )
            r.start()
            sent.append(r)
            return r

        desc = {}
        for b in range(B):
            desc["kcw", b, 0] = start(kcw_s, kcw_r, b, 0, ocw_ref[0], 0, right)
            desc["vcw", b, 0] = start(vcw_s, vcw_r, b, 0, ocw_ref[0], HD, right)
            desc["kccw", b, 0] = start(
                kccw_s, kccw_r, b, 0, occw_ref[0], 0, left)
            desc["vccw", b, 0] = start(
                vccw_s, vccw_r, b, 0, occw_ref[0], HD, left)

        wq = wq_ref[...].astype(jnp.bfloat16)
        qs = []
        for b in range(B):
            xb = x_ref[b].astype(jnp.bfloat16)
            q = rope(jnp.dot(xb, wq, preferred_element_type=jnp.float32))
            qs.append(q.astype(jnp.bfloat16))

        def score_update(origin):
            blk = pl.ds(origin * SQ, SQ)
            for b in range(B):
                for hh in range(HQ):
                    kb = kv_ref[b, blk, hh * DH:(hh + 1) * DH]
                    qbh = qs[b][:, hh * DH:(hh + 1) * DH]
                    s = lax.dot_general(
                        qbh, kb, (((1,), (1,)), ((), ())),
                        preferred_element_type=jnp.float32)
                    s_ref[b, hh, :, blk] = s

        score_update(ocw_ref[0])

        for h in range(CW):
            for b in range(B):
                desc["kcw", b, h].wait_recv()
                if h + 1 < CW:
                    desc["kcw", b, h + 1] = start(
                        kcw_s, kcw_r, b, h + 1, ocw_ref[h + 1], 0, right)
                desc["vcw", b, h].wait_recv()
                if h + 1 < CW:
                    desc["vcw", b, h + 1] = start(
                        vcw_s, vcw_r, b, h + 1, ocw_ref[h + 1], HD, right)
            if h < CCW:
                for b in range(B):
                    desc["kccw", b, h].wait_recv()
                    if h + 1 < CCW:
                        desc["kccw", b, h + 1] = start(
                            kccw_s, kccw_r, b, h + 1, occw_ref[h + 1], 0, left)
                    desc["vccw", b, h].wait_recv()
                    if h + 1 < CCW:
                        desc["vccw", b, h + 1] = start(
                            vccw_s, vccw_r, b, h + 1, occw_ref[h + 1], HD,
                            left)
            score_update(ocw_ref[h + 1])
            if h < CCW:
                score_update(occw_ref[h + 1])

        wo = wo_ref[...].astype(jnp.bfloat16)
        for b in range(B):
            ctx_heads = []
            for hh in range(HQ):
                s = s_ref[b, hh] * 0.125
                m = jnp.max(s, axis=1, keepdims=True)
                p = jnp.exp(s - m)
                l = jnp.sum(p, axis=1, keepdims=True)
                vf = kv_ref[b, :, HD + hh * DH:HD + (hh + 1) * DH]
                pv = lax.dot_general(
                    p.astype(jnp.bfloat16), vf, (((1,), (0,)), ((), ())),
                    preferred_element_type=jnp.float32)
                ctx_heads.append(pv / l)
            ctx_b = jnp.concatenate(ctx_heads, axis=1).astype(jnp.bfloat16)
            out_ref[b] = jnp.dot(ctx_b, wo, preferred_element_type=jnp.float32)

        for r in sent:
            r.wait_send()

        @functools.partial(pl.run_scoped, sem=pltpu.SemaphoreType.REGULAR)
        def _(sem):
            for nbr in (left, right):
                pl.semaphore_signal(sem, inc=1, device_id=(nbr,),
                                    device_id_type=pl.DeviceIdType.MESH)
            pl.semaphore_wait(sem, 2)

    return pl.pallas_call(
        body,
        out_shape=jax.ShapeDtypeStruct((B, SQ, D), jnp.float32),
        in_specs=(
            [pl.BlockSpec(memory_space=pltpu.VMEM)] * 5
            + [pl.BlockSpec(memory_space=pltpu.SMEM)] * 3
        ),
        out_specs=pl.BlockSpec(memory_space=pltpu.VMEM),
        scratch_shapes=[
            pltpu.VMEM((B, S_GLOBAL, 2 * HD), jnp.bfloat16),
            pltpu.VMEM((B, HQ, SQ, S_GLOBAL), jnp.float32),
            pltpu.SemaphoreType.DMA((B, CW)),
            pltpu.SemaphoreType.DMA((B, CW)),
            pltpu.SemaphoreType.DMA((B, CW)),
            pltpu.SemaphoreType.DMA((B, CW)),
            pltpu.SemaphoreType.DMA((B, CCW)),
            pltpu.SemaphoreType.DMA((B, CCW)),
            pltpu.SemaphoreType.DMA((B, CCW)),
            pltpu.SemaphoreType.DMA((B, CCW)),
        ],
        compiler_params=pltpu.CompilerParams(collective_id=0),
    )(x, Wq, Wk, Wv, Wo, nbrs, ocw, occw)


# device time: 72044 ns/iter; 2.7179x vs baseline; 1.0304x over previous
import functools

import jax
import jax.numpy as jnp
import numpy as np
from jax import lax
from jax.experimental import pallas as pl
from jax.experimental.pallas import tpu as pltpu

N_DEV = 32
B = 2
SQ = 128
D = 512
HQ = 4
DH = 64
HD = HQ * DH
S_GLOBAL = N_DEV * SQ
LCW = 8
LCCW = 7

_YZ_CYCLE = [
    (0, 0), (0, 1), (0, 2), (0, 3), (1, 3), (1, 2), (1, 1), (2, 1),
    (2, 2), (2, 3), (3, 3), (3, 2), (3, 1), (3, 0), (2, 0), (1, 0),
]


@functools.lru_cache(maxsize=1)
def _tables():
    import distributed_mesh_v7x as dm

    mesh = dm.get_mesh("i", N_DEV)
    devs = list(mesh.devices.flat)
    c2l = {tuple(d.coords): i for i, d in enumerate(devs)}
    xs = sorted({c[0] for c in c2l})
    ys = sorted({c[1] for c in c2l})
    zs = sorted({c[2] for c in c2l})
    assert len(xs) == 2 and len(ys) == 4 and len(zs) == 4, (xs, ys, zs)

    def lid(xi, yi, zi):
        return c2l[(xs[xi], ys[yi], zs[zi])]

    pos_in_cycle = {yz: q for q, yz in enumerate(_YZ_CYCLE)}
    nbrs = np.zeros((N_DEV, 3), np.int32)
    ocw = np.zeros((N_DEV, 2, LCW + 1), np.int32)
    occw = np.zeros((N_DEV, 2, LCCW + 1), np.int32)
    for coords, l in c2l.items():
        xi = xs.index(coords[0])
        yi = ys.index(coords[1])
        zi = zs.index(coords[2])
        q = pos_in_cycle[(yi, zi)]
        lft = _YZ_CYCLE[(q - 1) % 16]
        rgt = _YZ_CYCLE[(q + 1) % 16]
        nbrs[l] = (lid(xi, *lft), lid(xi, *rgt), lid(1 - xi, yi, zi))
        for h in range(LCW + 1):
            yz = _YZ_CYCLE[(q - h) % 16]
            ocw[l, 0, h] = lid(xi, *yz)
            ocw[l, 1, h] = lid(1 - xi, *yz)
        for h in range(LCCW + 1):
            yz = _YZ_CYCLE[(q + h) % 16]
            occw[l, 0, h] = lid(xi, *yz)
            occw[l, 1, h] = lid(1 - xi, *yz)
    return nbrs, ocw, occw


def kernel(x, Wq, Wk, Wv, Wo):
    nbrs_np, ocw_np, occw_np = _tables()
    my = lax.axis_index("i")
    nbrs = jnp.take(jnp.asarray(nbrs_np), my, axis=0)
    ocw = jnp.take(jnp.asarray(ocw_np), my, axis=0)
    occw = jnp.take(jnp.asarray(occw_np), my, axis=0)

    def body(x_ref, wq_ref, wk_ref, wv_ref, wo_ref, nbr_ref, ocw_ref,
             occw_ref, out_ref, kv_ref, s_ref,
             kx_s, kx_r, vx_s, vx_r,
             kcw_s, kcw_r, vcw_s, vcw_r, kccw_s, kccw_r, vccw_s, vccw_r):
        my = lax.axis_index("i")
        left = nbr_ref[0]
        right = nbr_ref[1]
        partner = nbr_ref[2]

        barrier = pltpu.get_barrier_semaphore()
        for nbr in (left, right, partner):
            pl.semaphore_signal(barrier, inc=1, device_id=(nbr,),
                                device_id_type=pl.DeviceIdType.MESH)
        pl.semaphore_wait(barrier, 3)

        d_idx = lax.broadcasted_iota(jnp.int32, (SQ, HD), 1)
        half = lax.rem(d_idx, DH) // 2
        inv = jnp.exp(half.astype(jnp.float32) * (-2.0 / DH * jnp.log(10000.0)))
        posn = lax.broadcasted_iota(jnp.int32, (SQ, HD), 0) + my * SQ
        ang = posn.astype(jnp.float32) * inv
        cos_t = jnp.cos(ang)
        sin_t = jnp.sin(ang)
        even = lax.rem(d_idx, 2) == 0

        def rope(t):
            t_rot = jnp.where(even, -jnp.roll(t, -1, axis=1),
                              jnp.roll(t, 1, axis=1))
            return t * cos_t + t_rot * sin_t

        wk = wk_ref[...].astype(jnp.bfloat16)
        wv = wv_ref[...].astype(jnp.bfloat16)
        for b in range(B):
            xb = x_ref[b].astype(jnp.bfloat16)
            k = rope(jnp.dot(xb, wk, preferred_element_type=jnp.float32))
            v = jnp.dot(xb, wv, preferred_element_type=jnp.float32)
            row = pl.ds(my * SQ, SQ)
            kv_ref[b, row, :HD] = k.astype(jnp.bfloat16)
            kv_ref[b, row, HD:] = v.astype(jnp.bfloat16)

        sent = []

        def rdma(origin, b, col_off, send_sem, recv_sem, dev):
            blk = pl.ds(origin * SQ, SQ)
            r = pltpu.make_async_remote_copy(
                src_ref=kv_ref.at[b, blk, pl.ds(col_off, HD)],
                dst_ref=kv_ref.at[b, blk, pl.ds(col_off, HD)],
                send_sem=send_sem,
                recv_sem=recv_sem,
                device_id=(dev,),
                device_id_type=pl.DeviceIdType.MESH,
            )
            r.start()
            sent.append(r)
            return r

        desc = {}
        for b in range(B):
            desc["kx", b] = rdma(my, b, 0, kx_s.at[b], kx_r.at[b], partner)
            desc["vx", b] = rdma(my, b, HD, vx_s.at[b], vx_r.at[b], partner)
            desc["kcw", b, 0] = rdma(
                my, b, 0, kcw_s.at[0, b, 0], kcw_r.at[0, b, 0], right)
            desc["vcw", b, 0] = rdma(
                my, b, HD, vcw_s.at[0, b, 0], vcw_r.at[0, b, 0], right)
            desc["kccw", b, 0] = rdma(
                my, b, 0, kccw_s.at[0, b, 0], kccw_r.at[0, b, 0], left)
            desc["vccw", b, 0] = rdma(
                my, b, HD, vccw_s.at[0, b, 0], vccw_r.at[0, b, 0], left)

        wq = wq_ref[...].astype(jnp.bfloat16)
        qs = []
        for b in range(B):
            xb = x_ref[b].astype(jnp.bfloat16)
            q = rope(jnp.dot(xb, wq, preferred_element_type=jnp.float32))
            qs.append(q.astype(jnp.bfloat16))

        def score_update(origin):
            blk = pl.ds(origin * SQ, SQ)
            for b in range(B):
                for hh in range(HQ):
                    kb = kv_ref[b, blk, hh * DH:(hh + 1) * DH]
                    qbh = qs[b][:, hh * DH:(hh + 1) * DH]
                    s = lax.dot_general(
                        qbh, kb, (((1,), (1,)), ((), ())),
                        preferred_element_type=jnp.float32)
                    s_ref[b, hh, :, blk] = s

        score_update(my)

        for b in range(B):
            desc["kx", b].wait_recv()
            desc["vx", b].wait_recv()
        score_update(partner)
        for b in range(B):
            desc["kcw2", b, 0] = rdma(
                partner, b, 0, kcw_s.at[1, b, 0], kcw_r.at[1, b, 0], right)
            desc["vcw2", b, 0] = rdma(
                partner, b, HD, vcw_s.at[1, b, 0], vcw_r.at[1, b, 0], right)
            desc["kccw2", b, 0] = rdma(
                partner, b, 0, kccw_s.at[1, b, 0], kccw_r.at[1, b, 0], left)
            desc["vccw2", b, 0] = rdma(
                partner, b, HD, vccw_s.at[1, b, 0], vccw_r.at[1, b, 0], left)

        for h in range(LCW):
            for b in range(B):
                desc["kcw", b, h].wait_recv()
                if h + 1 < LCW:
                    desc["kcw", b, h + 1] = rdma(
                        ocw_ref[0, h + 1], b, 0,
                        kcw_s.at[0, b, h + 1], kcw_r.at[0, b, h + 1], right)
                desc["vcw", b, h].wait_recv()
                if h + 1 < LCW:
                    desc["vcw", b, h + 1] = rdma(
                        ocw_ref[0, h + 1], b, HD,
                        vcw_s.at[0, b, h + 1], vcw_r.at[0, b, h + 1], right)
                desc["kcw2", b, h].wait_recv()
                if h + 1 < LCW:
                    desc["kcw2", b, h + 1] = rdma(
                        ocw_ref[1, h + 1], b, 0,
                        kcw_s.at[1, b, h + 1], kcw_r.at[1, b, h + 1], right)
                desc["vcw2", b, h].wait_recv()
                if h + 1 < LCW:
                    desc["vcw2", b, h + 1] = rdma(
                        ocw_ref[1, h + 1], b, HD,
                        vcw_s.at[1, b, h + 1], vcw_r.at[1, b, h + 1], right)
            if h < LCCW:
                for b in range(B):
                    desc["kccw", b, h].wait_recv()
                    if h + 1 < LCCW:
                        desc["kccw", b, h + 1] = rdma(
                            occw_ref[0, h + 1], b, 0,
                            kccw_s.at[0, b, h + 1], kccw_r.at[0, b, h + 1],
                            left)
                    desc["vccw", b, h].wait_recv()
                    if h + 1 < LCCW:
                        desc["vccw", b, h + 1] = rdma(
                            occw_ref[0, h + 1], b, HD,
                            vccw_s.at[0, b, h + 1], vccw_r.at[0, b, h + 1],
                            left)
                    desc["kccw2", b, h].wait_recv()
                    if h + 1 < LCCW:
                        desc["kccw2", b, h + 1] = rdma(
                            occw_ref[1, h + 1], b, 0,
                            kccw_s.at[1, b, h + 1], kccw_r.at[1, b, h + 1],
                            left)
                    desc["vccw2", b, h].wait_recv()
                    if h + 1 < LCCW:
                        desc["vccw2", b, h + 1] = rdma(
                            occw_ref[1, h + 1], b, HD,
                            vccw_s.at[1, b, h + 1], vccw_r.at[1, b, h + 1],
                            left)
            score_update(ocw_ref[0, h + 1])
            score_update(ocw_ref[1, h + 1])
            if h < LCCW:
                score_update(occw_ref[0, h + 1])
                score_update(occw_ref[1, h + 1])

        wo = wo_ref[...].astype(jnp.bfloat16)
        for b in range(B):
            ctx_heads = []
            for hh in range(HQ):
                s = s_ref[b, hh] * 0.125
                m = jnp.max(s, axis=1, keepdims=True)
                p = jnp.exp(s - m)
                l = jnp.sum(p, axis=1, keepdims=True)
                vf = kv_ref[b, :, HD + hh * DH:HD + (hh + 1) * DH]
                pv = lax.dot_general(
                    p.astype(jnp.bfloat16), vf, (((1,), (0,)), ((), ())),
                    preferred_element_type=jnp.float32)
                ctx_heads.append(pv / l)
            ctx_b = jnp.concatenate(ctx_heads, axis=1).astype(jnp.bfloat16)
            out_ref[b] = jnp.dot(ctx_b, wo, preferred_element_type=jnp.float32)

        for r in sent:
            r.wait_send()

        @functools.partial(pl.run_scoped, sem=pltpu.SemaphoreType.REGULAR)
        def _(sem):
            for nbr in (left, right, partner):
                pl.semaphore_signal(sem, inc=1, device_id=(nbr,),
                                    device_id_type=pl.DeviceIdType.MESH)
            pl.semaphore_wait(sem, 3)

    return pl.pallas_call(
        body,
        out_shape=jax.ShapeDtypeStruct((B, SQ, D), jnp.float32),
        in_specs=(
            [pl.BlockSpec(memory_space=pltpu.VMEM)] * 5
            + [pl.BlockSpec(memory_space=pltpu.SMEM)] * 3
        ),
        out_specs=pl.BlockSpec(memory_space=pltpu.VMEM),
        scratch_shapes=[
            pltpu.VMEM((B, S_GLOBAL, 2 * HD), jnp.bfloat16),
            pltpu.VMEM((B, HQ, SQ, S_GLOBAL), jnp.float32),
            pltpu.SemaphoreType.DMA((B,)),
            pltpu.SemaphoreType.DMA((B,)),
            pltpu.SemaphoreType.DMA((B,)),
            pltpu.SemaphoreType.DMA((B,)),
            pltpu.SemaphoreType.DMA((2, B, LCW)),
            pltpu.SemaphoreType.DMA((2, B, LCW)),
            pltpu.SemaphoreType.DMA((2, B, LCW)),
            pltpu.SemaphoreType.DMA((2, B, LCW)),
            pltpu.SemaphoreType.DMA((2, B, LCCW)),
            pltpu.SemaphoreType.DMA((2, B, LCCW)),
            pltpu.SemaphoreType.DMA((2, B, LCCW)),
            pltpu.SemaphoreType.DMA((2, B, LCCW)),
        ],
        compiler_params=pltpu.CompilerParams(collective_id=0),
    )(x, Wq, Wk, Wv, Wo, nbrs, ocw.reshape(2, LCW + 1),
      occw.reshape(2, LCCW + 1))


# device time: 71064 ns/iter; 2.7553x vs baseline; 1.0138x over previous
import functools

import jax
import jax.numpy as jnp
import numpy as np
from jax import lax
from jax.experimental import pallas as pl
from jax.experimental.pallas import tpu as pltpu

N_DEV = 32
B = 2
SQ = 128
D = 512
HQ = 4
DH = 64
HD = HQ * DH
S_GLOBAL = N_DEV * SQ
LCW = 8
LCCW = 7

_YZ_CYCLE = [
    (0, 0), (0, 1), (0, 2), (0, 3), (1, 3), (1, 2), (1, 1), (2, 1),
    (2, 2), (2, 3), (3, 3), (3, 2), (3, 1), (3, 0), (2, 0), (1, 0),
]


@functools.lru_cache(maxsize=1)
def _tables():
    import distributed_mesh_v7x as dm

    mesh = dm.get_mesh("i", N_DEV)
    devs = list(mesh.devices.flat)
    c2l = {tuple(d.coords): i for i, d in enumerate(devs)}
    xs = sorted({c[0] for c in c2l})
    ys = sorted({c[1] for c in c2l})
    zs = sorted({c[2] for c in c2l})
    assert len(xs) == 2 and len(ys) == 4 and len(zs) == 4, (xs, ys, zs)

    def lid(xi, yi, zi):
        return c2l[(xs[xi], ys[yi], zs[zi])]

    pos_in_cycle = {yz: q for q, yz in enumerate(_YZ_CYCLE)}
    nbrs = np.zeros((N_DEV, 3), np.int32)
    ocw = np.zeros((N_DEV, 2, LCW + 1), np.int32)
    occw = np.zeros((N_DEV, 2, LCCW + 1), np.int32)
    for coords, l in c2l.items():
        xi = xs.index(coords[0])
        yi = ys.index(coords[1])
        zi = zs.index(coords[2])
        q = pos_in_cycle[(yi, zi)]
        lft = _YZ_CYCLE[(q - 1) % 16]
        rgt = _YZ_CYCLE[(q + 1) % 16]
        nbrs[l] = (lid(xi, *lft), lid(xi, *rgt), lid(1 - xi, yi, zi))
        for h in range(LCW + 1):
            yz = _YZ_CYCLE[(q - h) % 16]
            ocw[l, 0, h] = lid(xi, *yz)
            ocw[l, 1, h] = lid(1 - xi, *yz)
        for h in range(LCCW + 1):
            yz = _YZ_CYCLE[(q + h) % 16]
            occw[l, 0, h] = lid(xi, *yz)
            occw[l, 1, h] = lid(1 - xi, *yz)
    return nbrs, ocw, occw


def kernel(x, Wq, Wk, Wv, Wo):
    nbrs_np, ocw_np, occw_np = _tables()
    my = lax.axis_index("i")
    nbrs = jnp.take(jnp.asarray(nbrs_np), my, axis=0)
    ocw = jnp.take(jnp.asarray(ocw_np), my, axis=0)
    occw = jnp.take(jnp.asarray(occw_np), my, axis=0)

    def body(x_ref, wq_ref, wk_ref, wv_ref, wo_ref, nbr_ref, ocw_ref,
             occw_ref, out_ref, kv_ref, s_ref,
             kx_s, kx_r, vx_s, vx_r,
             kcw_s, kcw_r, vcw_s, vcw_r, kccw_s, kccw_r, vccw_s, vccw_r):
        my = lax.axis_index("i")
        left = nbr_ref[0]
        right = nbr_ref[1]
        partner = nbr_ref[2]

        barrier = pltpu.get_barrier_semaphore()
        for nbr in (left, right, partner):
            pl.semaphore_signal(barrier, inc=1, device_id=(nbr,),
                                device_id_type=pl.DeviceIdType.MESH)
        pl.semaphore_wait(barrier, 3)

        d_idx = lax.broadcasted_iota(jnp.int32, (SQ, HD), 1)
        half = lax.rem(d_idx, DH) // 2
        inv = jnp.exp(half.astype(jnp.float32) * (-2.0 / DH * jnp.log(10000.0)))
        posn = lax.broadcasted_iota(jnp.int32, (SQ, HD), 0) + my * SQ
        ang = posn.astype(jnp.float32) * inv
        cos_t = jnp.cos(ang)
        sin_t = jnp.sin(ang)
        even = lax.rem(d_idx, 2) == 0

        def rope(t):
            t_rot = jnp.where(even, -jnp.roll(t, -1, axis=1),
                              jnp.roll(t, 1, axis=1))
            return t * cos_t + t_rot * sin_t

        sent = []

        def rdma(origin, b, col_off, send_sem, recv_sem, dev):
            blk = pl.ds(origin * SQ, SQ)
            r = pltpu.make_async_remote_copy(
                src_ref=kv_ref.at[b, blk, pl.ds(col_off, HD)],
                dst_ref=kv_ref.at[b, blk, pl.ds(col_off, HD)],
                send_sem=send_sem,
                recv_sem=recv_sem,
                device_id=(dev,),
                device_id_type=pl.DeviceIdType.MESH,
            )
            r.start()
            sent.append(r)
            return r

        wk = wk_ref[...].astype(jnp.bfloat16)
        wv = wv_ref[...].astype(jnp.bfloat16)
        desc = {}
        row = pl.ds(my * SQ, SQ)
        for b in range(B):
            xb = x_ref[b].astype(jnp.bfloat16)
            k = rope(jnp.dot(xb, wk, preferred_element_type=jnp.float32))
            kv_ref[b, row, :HD] = k.astype(jnp.bfloat16)
            desc["kx", b] = rdma(my, b, 0, kx_s.at[b], kx_r.at[b], partner)
            desc["kcw", b, 0] = rdma(
                my, b, 0, kcw_s.at[0, b, 0], kcw_r.at[0, b, 0], right)
            desc["kccw", b, 0] = rdma(
                my, b, 0, kccw_s.at[0, b, 0], kccw_r.at[0, b, 0], left)
            v = jnp.dot(xb, wv, preferred_element_type=jnp.float32)
            kv_ref[b, row, HD:] = v.astype(jnp.bfloat16)
            desc["vx", b] = rdma(my, b, HD, vx_s.at[b], vx_r.at[b], partner)
            desc["vcw", b, 0] = rdma(
                my, b, HD, vcw_s.at[0, b, 0], vcw_r.at[0, b, 0], right)
            desc["vccw", b, 0] = rdma(
                my, b, HD, vccw_s.at[0, b, 0], vccw_r.at[0, b, 0], left)

        wq = wq_ref[...].astype(jnp.bfloat16)
        qs = []
        for b in range(B):
            xb = x_ref[b].astype(jnp.bfloat16)
            q = rope(jnp.dot(xb, wq, preferred_element_type=jnp.float32))
            qs.append((q * 0.125).astype(jnp.bfloat16))

        def score_update(origin):
            blk = pl.ds(origin * SQ, SQ)
            for b in range(B):
                for hh in range(HQ):
                    kb = kv_ref[b, blk, hh * DH:(hh + 1) * DH]
                    qbh = qs[b][:, hh * DH:(hh + 1) * DH]
                    s = lax.dot_general(
                        qbh, kb, (((1,), (1,)), ((), ())),
                        preferred_element_type=jnp.float32)
                    s_ref[b, hh, :, blk] = s

        score_update(my)

        for b in range(B):
            desc["kx", b].wait_recv()
            desc["vx", b].wait_recv()
        score_update(partner)
        for b in range(B):
            desc["kcw2", b, 0] = rdma(
                partner, b, 0, kcw_s.at[1, b, 0], kcw_r.at[1, b, 0], right)
            desc["vcw2", b, 0] = rdma(
                partner, b, HD, vcw_s.at[1, b, 0], vcw_r.at[1, b, 0], right)
            desc["kccw2", b, 0] = rdma(
                partner, b, 0, kccw_s.at[1, b, 0], kccw_r.at[1, b, 0], left)
            desc["vccw2", b, 0] = rdma(
                partner, b, HD, vccw_s.at[1, b, 0], vccw_r.at[1, b, 0], left)

        for h in range(LCW):
            for b in range(B):
                desc["kcw", b, h].wait_recv()
                if h + 1 < LCW:
                    desc["kcw", b, h + 1] = rdma(
                        ocw_ref[0, h + 1], b, 0,
                        kcw_s.at[0, b, h + 1], kcw_r.at[0, b, h + 1], right)
                desc["vcw", b, h].wait_recv()
                if h + 1 < LCW:
                    desc["vcw", b, h + 1] = rdma(
                        ocw_ref[0, h + 1], b, HD,
                        vcw_s.at[0, b, h + 1], vcw_r.at[0, b, h + 1], right)
                desc["kcw2", b, h].wait_recv()
                if h + 1 < LCW:
                    desc["kcw2", b, h + 1] = rdma(
                        ocw_ref[1, h + 1], b, 0,
                        kcw_s.at[1, b, h + 1], kcw_r.at[1, b, h + 1], right)
                desc["vcw2", b, h].wait_recv()
                if h + 1 < LCW:
                    desc["vcw2", b, h + 1] = rdma(
                        ocw_ref[1, h + 1], b, HD,
                        vcw_s.at[1, b, h + 1], vcw_r.at[1, b, h + 1], right)
            if h < LCCW:
                for b in range(B):
                    desc["kccw", b, h].wait_recv()
                    if h + 1 < LCCW:
                        desc["kccw", b, h + 1] = rdma(
                            occw_ref[0, h + 1], b, 0,
                            kccw_s.at[0, b, h + 1], kccw_r.at[0, b, h + 1],
                            left)
                    desc["vccw", b, h].wait_recv()
                    if h + 1 < LCCW:
                        desc["vccw", b, h + 1] = rdma(
                            occw_ref[0, h + 1], b, HD,
                            vccw_s.at[0, b, h + 1], vccw_r.at[0, b, h + 1],
                            left)
                    desc["kccw2", b, h].wait_recv()
                    if h + 1 < LCCW:
                        desc["kccw2", b, h + 1] = rdma(
                            occw_ref[1, h + 1], b, 0,
                            kccw_s.at[1, b, h + 1], kccw_r.at[1, b, h + 1],
                            left)
                    desc["vccw2", b, h].wait_recv()
                    if h + 1 < LCCW:
                        desc["vccw2", b, h + 1] = rdma(
                            occw_ref[1, h + 1], b, HD,
                            vccw_s.at[1, b, h + 1], vccw_r.at[1, b, h + 1],
                            left)
            score_update(ocw_ref[0, h + 1])
            score_update(ocw_ref[1, h + 1])
            if h < LCCW:
                score_update(occw_ref[0, h + 1])
                score_update(occw_ref[1, h + 1])

        wo = wo_ref[...].astype(jnp.bfloat16)
        for b in range(B):
            ctx_heads = []
            for hh in range(HQ):
                s = s_ref[b, hh]
                m = jnp.max(s, axis=1, keepdims=True)
                p = jnp.exp(s - m)
                l = jnp.sum(p, axis=1, keepdims=True)
                vf = kv_ref[b, :, HD + hh * DH:HD + (hh + 1) * DH]
                pv = lax.dot_general(
                    p.astype(jnp.bfloat16), vf, (((1,), (0,)), ((), ())),
                    preferred_element_type=jnp.float32)
                ctx_heads.append(pv / l)
            ctx_b = jnp.concatenate(ctx_heads, axis=1).astype(jnp.bfloat16)
            out_ref[b] = jnp.dot(ctx_b, wo, preferred_element_type=jnp.float32)

        for r in sent:
            r.wait_send()

        @functools.partial(pl.run_scoped, sem=pltpu.SemaphoreType.REGULAR)
        def _(sem):
            for nbr in (left, right, partner):
                pl.semaphore_signal(sem, inc=1, device_id=(nbr,),
                                    device_id_type=pl.DeviceIdType.MESH)
            pl.semaphore_wait(sem, 3)

    return pl.pallas_call(
        body,
        out_shape=jax.ShapeDtypeStruct((B, SQ, D), jnp.float32),
        in_specs=(
            [pl.BlockSpec(memory_space=pltpu.VMEM)] * 5
            + [pl.BlockSpec(memory_space=pltpu.SMEM)] * 3
        ),
        out_specs=pl.BlockSpec(memory_space=pltpu.VMEM),
        scratch_shapes=[
            pltpu.VMEM((B, S_GLOBAL, 2 * HD), jnp.bfloat16),
            pltpu.VMEM((B, HQ, SQ, S_GLOBAL), jnp.float32),
            pltpu.SemaphoreType.DMA((B,)),
            pltpu.SemaphoreType.DMA((B,)),
            pltpu.SemaphoreType.DMA((B,)),
            pltpu.SemaphoreType.DMA((B,)),
            pltpu.SemaphoreType.DMA((2, B, LCW)),
            pltpu.SemaphoreType.DMA((2, B, LCW)),
            pltpu.SemaphoreType.DMA((2, B, LCW)),
            pltpu.SemaphoreType.DMA((2, B, LCW)),
            pltpu.SemaphoreType.DMA((2, B, LCCW)),
            pltpu.SemaphoreType.DMA((2, B, LCCW)),
            pltpu.SemaphoreType.DMA((2, B, LCCW)),
            pltpu.SemaphoreType.DMA((2, B, LCCW)),
        ],
        compiler_params=pltpu.CompilerParams(collective_id=0),
    )(x, Wq, Wk, Wv, Wo, nbrs, ocw.reshape(2, LCW + 1),
      occw.reshape(2, LCCW + 1))


# device time: 70607 ns/iter; 2.7732x vs baseline; 1.0065x over previous
import functools

import jax
import jax.numpy as jnp
import numpy as np
from jax import lax
from jax.experimental import pallas as pl
from jax.experimental.pallas import tpu as pltpu

N_DEV = 32
B = 2
SQ = 128
D = 512
HQ = 4
DH = 64
HD = HQ * DH
S_GLOBAL = N_DEV * SQ
LCW = 8
LCCW = 7

_YZ_CYCLE = [
    (0, 0), (0, 1), (0, 2), (0, 3), (1, 3), (1, 2), (1, 1), (2, 1),
    (2, 2), (2, 3), (3, 3), (3, 2), (3, 1), (3, 0), (2, 0), (1, 0),
]


@functools.lru_cache(maxsize=1)
def _tables():
    import distributed_mesh_v7x as dm

    mesh = dm.get_mesh("i", N_DEV)
    devs = list(mesh.devices.flat)
    c2l = {tuple(d.coords): i for i, d in enumerate(devs)}
    xs = sorted({c[0] for c in c2l})
    ys = sorted({c[1] for c in c2l})
    zs = sorted({c[2] for c in c2l})
    assert len(xs) == 2 and len(ys) == 4 and len(zs) == 4, (xs, ys, zs)

    def lid(xi, yi, zi):
        return c2l[(xs[xi], ys[yi], zs[zi])]

    pos_in_cycle = {yz: q for q, yz in enumerate(_YZ_CYCLE)}
    nbrs = np.zeros((N_DEV, 3), np.int32)
    ocw = np.zeros((N_DEV, 2, LCW + 1), np.int32)
    occw = np.zeros((N_DEV, 2, LCCW + 1), np.int32)
    for coords, l in c2l.items():
        xi = xs.index(coords[0])
        yi = ys.index(coords[1])
        zi = zs.index(coords[2])
        q = pos_in_cycle[(yi, zi)]
        lft = _YZ_CYCLE[(q - 1) % 16]
        rgt = _YZ_CYCLE[(q + 1) % 16]
        nbrs[l] = (lid(xi, *lft), lid(xi, *rgt), lid(1 - xi, yi, zi))
        for h in range(LCW + 1):
            yz = _YZ_CYCLE[(q - h) % 16]
            ocw[l, 0, h] = lid(xi, *yz)
            ocw[l, 1, h] = lid(1 - xi, *yz)
        for h in range(LCCW + 1):
            yz = _YZ_CYCLE[(q + h) % 16]
            occw[l, 0, h] = lid(xi, *yz)
            occw[l, 1, h] = lid(1 - xi, *yz)
    return nbrs, ocw, occw


def kernel(x, Wq, Wk, Wv, Wo):
    nbrs_np, ocw_np, occw_np = _tables()
    my = lax.axis_index("i")
    nbrs = jnp.take(jnp.asarray(nbrs_np), my, axis=0)
    ocw = jnp.take(jnp.asarray(ocw_np), my, axis=0)
    occw = jnp.take(jnp.asarray(occw_np), my, axis=0)

    def body(x_ref, wq_ref, wk_ref, wv_ref, wo_ref, nbr_ref, ocw_ref,
             occw_ref, out_ref, kv_ref, s_ref,
             kx_s, kx_r, vx_s, vx_r,
             kcw_s, kcw_r, vcw_s, vcw_r, kccw_s, kccw_r, vccw_s, vccw_r):
        my = lax.axis_index("i")
        left = nbr_ref[0]
        right = nbr_ref[1]
        partner = nbr_ref[2]

        barrier = pltpu.get_barrier_semaphore()
        for nbr in (left, right, partner):
            pl.semaphore_signal(barrier, inc=1, device_id=(nbr,),
                                device_id_type=pl.DeviceIdType.MESH)
        pl.semaphore_wait(barrier, 3)

        d_idx = lax.broadcasted_iota(jnp.int32, (SQ, HD), 1)
        half = lax.rem(d_idx, DH) // 2
        inv = jnp.exp(half.astype(jnp.float32) * (-2.0 / DH * jnp.log(10000.0)))
        posn = lax.broadcasted_iota(jnp.int32, (SQ, HD), 0) + my * SQ
        ang = posn.astype(jnp.float32) * inv
        cos_t = jnp.cos(ang)
        sin_t = jnp.sin(ang)
        even = lax.rem(d_idx, 2) == 0

        def rope(t):
            t_rot = jnp.where(even, -jnp.roll(t, -1, axis=1),
                              jnp.roll(t, 1, axis=1))
            return t * cos_t + t_rot * sin_t

        sent = []

        def rdma(origin, b, col_off, send_sem, recv_sem, dev):
            blk = pl.ds(origin * SQ, SQ)
            r = pltpu.make_async_remote_copy(
                src_ref=kv_ref.at[b, blk, pl.ds(col_off, HD)],
                dst_ref=kv_ref.at[b, blk, pl.ds(col_off, HD)],
                send_sem=send_sem,
                recv_sem=recv_sem,
                device_id=(dev,),
                device_id_type=pl.DeviceIdType.MESH,
            )
            r.start()
            sent.append(r)
            return r

        wk = wk_ref[...].astype(jnp.bfloat16)
        wv = wv_ref[...].astype(jnp.bfloat16)
        desc = {}
        row = pl.ds(my * SQ, SQ)
        for b in range(B):
            xb = x_ref[b].astype(jnp.bfloat16)
            k = rope(jnp.dot(xb, wk, preferred_element_type=jnp.float32))
            kv_ref[b, row, :HD] = k.astype(jnp.bfloat16)
            desc["kx", b] = rdma(my, b, 0, kx_s.at[b], kx_r.at[b], partner)
            desc["kcw", b, 0] = rdma(
                my, b, 0, kcw_s.at[0, b, 0], kcw_r.at[0, b, 0], right)
            desc["kccw", b, 0] = rdma(
                my, b, 0, kccw_s.at[0, b, 0], kccw_r.at[0, b, 0], left)
            v = jnp.dot(xb, wv, preferred_element_type=jnp.float32)
            kv_ref[b, row, HD:] = v.astype(jnp.bfloat16)
            desc["vx", b] = rdma(my, b, HD, vx_s.at[b], vx_r.at[b], partner)
            desc["vcw", b, 0] = rdma(
                my, b, HD, vcw_s.at[0, b, 0], vcw_r.at[0, b, 0], right)
            desc["vccw", b, 0] = rdma(
                my, b, HD, vccw_s.at[0, b, 0], vccw_r.at[0, b, 0], left)

        wq = wq_ref[...].astype(jnp.bfloat16)
        qs = []
        for b in range(B):
            xb = x_ref[b].astype(jnp.bfloat16)
            q = rope(jnp.dot(xb, wq, preferred_element_type=jnp.float32))
            qs.append((q * 0.125).astype(jnp.bfloat16))

        run_m = {(b, hh): jnp.full((SQ, 1), -1e30, jnp.float32)
                 for b in range(B) for hh in range(HQ)}

        def score_update(origin):
            blk = pl.ds(origin * SQ, SQ)
            for b in range(B):
                for hh in range(HQ):
                    kb = kv_ref[b, blk, hh * DH:(hh + 1) * DH]
                    qbh = qs[b][:, hh * DH:(hh + 1) * DH]
                    s = lax.dot_general(
                        qbh, kb, (((1,), (1,)), ((), ())),
                        preferred_element_type=jnp.float32)
                    s_ref[b, hh, :, blk] = s
                    run_m[b, hh] = jnp.maximum(
                        run_m[b, hh], jnp.max(s, axis=1, keepdims=True))

        score_update(my)

        for b in range(B):
            desc["kx", b].wait_recv()
            desc["vx", b].wait_recv()
        score_update(partner)
        for b in range(B):
            desc["kcw2", b, 0] = rdma(
                partner, b, 0, kcw_s.at[1, b, 0], kcw_r.at[1, b, 0], right)
            desc["vcw2", b, 0] = rdma(
                partner, b, HD, vcw_s.at[1, b, 0], vcw_r.at[1, b, 0], right)
            desc["kccw2", b, 0] = rdma(
                partner, b, 0, kccw_s.at[1, b, 0], kccw_r.at[1, b, 0], left)
            desc["vccw2", b, 0] = rdma(
                partner, b, HD, vccw_s.at[1, b, 0], vccw_r.at[1, b, 0], left)

        for h in range(LCW):
            for b in range(B):
                desc["kcw", b, h].wait_recv()
                if h + 1 < LCW:
                    desc["kcw", b, h + 1] = rdma(
                        ocw_ref[0, h + 1], b, 0,
                        kcw_s.at[0, b, h + 1], kcw_r.at[0, b, h + 1], right)
                desc["vcw", b, h].wait_recv()
                if h + 1 < LCW:
                    desc["vcw", b, h + 1] = rdma(
                        ocw_ref[0, h + 1], b, HD,
                        vcw_s.at[0, b, h + 1], vcw_r.at[0, b, h + 1], right)
                desc["kcw2", b, h].wait_recv()
                if h + 1 < LCW:
                    desc["kcw2", b, h + 1] = rdma(
                        ocw_ref[1, h + 1], b, 0,
                        kcw_s.at[1, b, h + 1], kcw_r.at[1, b, h + 1], right)
                desc["vcw2", b, h].wait_recv()
                if h + 1 < LCW:
                    desc["vcw2", b, h + 1] = rdma(
                        ocw_ref[1, h + 1], b, HD,
                        vcw_s.at[1, b, h + 1], vcw_r.at[1, b, h + 1], right)
            if h < LCCW:
                for b in range(B):
                    desc["kccw", b, h].wait_recv()
                    if h + 1 < LCCW:
                        desc["kccw", b, h + 1] = rdma(
                            occw_ref[0, h + 1], b, 0,
                            kccw_s.at[0, b, h + 1], kccw_r.at[0, b, h + 1],
                            left)
                    desc["vccw", b, h].wait_recv()
                    if h + 1 < LCCW:
                        desc["vccw", b, h + 1] = rdma(
                            occw_ref[0, h + 1], b, HD,
                            vccw_s.at[0, b, h + 1], vccw_r.at[0, b, h + 1],
                            left)
                    desc["kccw2", b, h].wait_recv()
                    if h + 1 < LCCW:
                        desc["kccw2", b, h + 1] = rdma(
                            occw_ref[1, h + 1], b, 0,
                            kccw_s.at[1, b, h + 1], kccw_r.at[1, b, h + 1],
                            left)
                    desc["vccw2", b, h].wait_recv()
                    if h + 1 < LCCW:
                        desc["vccw2", b, h + 1] = rdma(
                            occw_ref[1, h + 1], b, HD,
                            vccw_s.at[1, b, h + 1], vccw_r.at[1, b, h + 1],
                            left)
            score_update(ocw_ref[0, h + 1])
            score_update(ocw_ref[1, h + 1])
            if h < LCCW:
                score_update(occw_ref[0, h + 1])
                score_update(occw_ref[1, h + 1])

        wo = wo_ref[...].astype(jnp.bfloat16)
        for b in range(B):
            ctx_heads = []
            for hh in range(HQ):
                s = s_ref[b, hh]
                m = run_m[b, hh]
                p = jnp.exp(s - m)
                l = jnp.sum(p, axis=1, keepdims=True)
                vf = kv_ref[b, :, HD + hh * DH:HD + (hh + 1) * DH]
                pv = lax.dot_general(
                    p.astype(jnp.bfloat16), vf, (((1,), (0,)), ((), ())),
                    preferred_element_type=jnp.float32)
                ctx_heads.append(pv / l)
            ctx_b = jnp.concatenate(ctx_heads, axis=1).astype(jnp.bfloat16)
            out_ref[b] = jnp.dot(ctx_b, wo, preferred_element_type=jnp.float32)

        for r in sent:
            r.wait_send()

        @functools.partial(pl.run_scoped, sem=pltpu.SemaphoreType.REGULAR)
        def _(sem):
            for nbr in (left, right, partner):
                pl.semaphore_signal(sem, inc=1, device_id=(nbr,),
                                    device_id_type=pl.DeviceIdType.MESH)
            pl.semaphore_wait(sem, 3)

    return pl.pallas_call(
        body,
        out_shape=jax.ShapeDtypeStruct((B, SQ, D), jnp.float32),
        in_specs=(
            [pl.BlockSpec(memory_space=pltpu.VMEM)] * 5
            + [pl.BlockSpec(memory_space=pltpu.SMEM)] * 3
        ),
        out_specs=pl.BlockSpec(memory_space=pltpu.VMEM),
        scratch_shapes=[
            pltpu.VMEM((B, S_GLOBAL, 2 * HD), jnp.bfloat16),
            pltpu.VMEM((B, HQ, SQ, S_GLOBAL), jnp.float32),
            pltpu.SemaphoreType.DMA((B,)),
            pltpu.SemaphoreType.DMA((B,)),
            pltpu.SemaphoreType.DMA((B,)),
            pltpu.SemaphoreType.DMA((B,)),
            pltpu.SemaphoreType.DMA((2, B, LCW)),
            pltpu.SemaphoreType.DMA((2, B, LCW)),
            pltpu.SemaphoreType.DMA((2, B, LCW)),
            pltpu.SemaphoreType.DMA((2, B, LCW)),
            pltpu.SemaphoreType.DMA((2, B, LCCW)),
            pltpu.SemaphoreType.DMA((2, B, LCCW)),
            pltpu.SemaphoreType.DMA((2, B, LCCW)),
            pltpu.SemaphoreType.DMA((2, B, LCCW)),
        ],
        compiler_params=pltpu.CompilerParams(collective_id=0),
    )(x, Wq, Wk, Wv, Wo, nbrs, ocw.reshape(2, LCW + 1),
      occw.reshape(2, LCCW + 1))
